# baseline (speedup 1.0000x reference)
import functools

import jax
import jax.numpy as jnp
from jax.experimental import pallas as pl
from jax.experimental.pallas import tpu as pltpu

NS_ITERS = 5
EPS = 1e-05
NCORES = 2


def _stats_kernel(x_ref, gram_ref, sum_ref):
    j = pl.program_id(1)

    @pl.when(j == 0)
    def _init():
        gram_ref[0] = jnp.zeros_like(gram_ref[0])
        sum_ref[0] = jnp.zeros_like(sum_ref[0])

    gram = gram_ref[0]
    ssum = sum_ref[0]
    for r in range(x_ref.shape[0]):
        x = x_ref[r]
        gram += jax.lax.dot_general(
            x, x, (((1,), (1,)), ((), ())), preferred_element_type=jnp.float32
        )
        ssum += jnp.sum(x, axis=1, keepdims=True)
    gram_ref[0] = gram
    sum_ref[0] = ssum


def _apply_kernel(m_total, gram_ref, sum_ref, x_ref, o_ref, wm_ref, wb_ref):
    j = pl.program_id(1)

    @pl.when(j == 0)
    def _compute_wm():
        d = gram_ref.shape[1]
        gram = gram_ref[0] + gram_ref[1]
        s = sum_ref[0] + sum_ref[1]
        inv_m = 1.0 / jnp.float32(m_total)
        mean = s * inv_m
        rows = jax.lax.broadcasted_iota(jnp.int32, (d, d), 0)
        cols = jax.lax.broadcasted_iota(jnp.int32, (d, d), 1)
        eye = jnp.where(rows == cols, jnp.float32(1.0), jnp.float32(0.0))
        outer = jax.lax.dot_general(
            mean, mean, (((1,), (1,)), ((), ())),
            preferred_element_type=jnp.float32,
        )
        sigma = gram * inv_m - outer + EPS * eye
        tr = jnp.sum(jnp.where(rows == cols, sigma, jnp.float32(0.0)))
        r_tr = 1.0 / tr
        sigma_n = sigma * r_tr
        p = eye
        for _ in range(NS_ITERS):
            p2 = jnp.dot(p, p, preferred_element_type=jnp.float32)
            ps = jnp.dot(p, sigma_n, preferred_element_type=jnp.float32)
            p = 1.5 * p - 0.5 * jnp.dot(
                p2, ps, preferred_element_type=jnp.float32
            )
        wm = p * jnp.sqrt(r_tr)
        wm_ref[...] = wm
        wb_ref[...] = jnp.dot(wm, mean, preferred_element_type=jnp.float32)

    for r in range(x_ref.shape[0]):
        o_ref[r] = (
            jnp.dot(wm_ref[...], x_ref[r], preferred_element_type=jnp.float32)
            - wb_ref[...]
        )


def kernel(X):
    B, C, L = X.shape
    m_total = B * L
    bb = 4
    blocks_per_core = B // (NCORES * bb)

    grid = (NCORES, blocks_per_core)
    x_spec = pl.BlockSpec(
        (bb, C, L), lambda i, j, nb=blocks_per_core: (i * nb + j, 0, 0)
    )

    sb = 8
    stats_blocks = B // (NCORES * sb)
    gram_p, sum_p = pl.pallas_call(
        _stats_kernel,
        grid=(NCORES, stats_blocks),
        in_specs=[
            pl.BlockSpec(
                (sb, C, L), lambda i, j, nb=stats_blocks: (i * nb + j, 0, 0)
            )
        ],
        out_specs=[
            pl.BlockSpec((1, C, C), lambda i, j: (i, 0, 0)),
            pl.BlockSpec((1, C, 1), lambda i, j: (i, 0, 0)),
        ],
        out_shape=[
            jax.ShapeDtypeStruct((NCORES, C, C), jnp.float32),
            jax.ShapeDtypeStruct((NCORES, C, 1), jnp.float32),
        ],
        compiler_params=pltpu.CompilerParams(
            dimension_semantics=("parallel", "arbitrary"),
            vmem_limit_bytes=56 * 1024 * 1024,
        ),
        name="iternorm_stats",
    )(X)

    out = pl.pallas_call(
        functools.partial(_apply_kernel, m_total),
        grid=grid,
        in_specs=[
            pl.BlockSpec((NCORES, C, C), lambda i, j: (0, 0, 0)),
            pl.BlockSpec((NCORES, C, 1), lambda i, j: (0, 0, 0)),
            x_spec,
        ],
        out_specs=x_spec,
        out_shape=jax.ShapeDtypeStruct((B, C, L), jnp.float32),
        scratch_shapes=[
            pltpu.VMEM((C, C), jnp.float32),
            pltpu.VMEM((C, 1), jnp.float32),
        ],
        compiler_params=pltpu.CompilerParams(
            dimension_semantics=("parallel", "arbitrary"),
            vmem_limit_bytes=56 * 1024 * 1024,
        ),
        name="iternorm_apply",
    )(gram_p, sum_p, X)

    return out


# speedup vs baseline: 1.0609x; 1.0609x over previous
import functools

import jax
import jax.numpy as jnp
from jax.experimental import pallas as pl
from jax.experimental.pallas import tpu as pltpu

NS_ITERS = 5
EPS = 1e-05


def _fused_kernel(m_total, nb, x_ref, o_ref, gram_ref, sum_ref, wm_ref, wb_ref):
    j = pl.program_id(0)

    @pl.when(j == 0)
    def _init():
        gram_ref[...] = jnp.zeros_like(gram_ref)
        sum_ref[...] = jnp.zeros_like(sum_ref)

    @pl.when(j < nb)
    def _stats():
        gram = gram_ref[...]
        ssum = sum_ref[...]
        for r in range(x_ref.shape[0]):
            x = x_ref[r]
            gram += jax.lax.dot_general(
                x, x, (((1,), (1,)), ((), ())),
                preferred_element_type=jnp.float32,
            )
            ssum += jnp.sum(x, axis=1, keepdims=True)
        gram_ref[...] = gram
        sum_ref[...] = ssum

    @pl.when(j == nb)
    def _compute_wm():
        d = gram_ref.shape[0]
        gram = gram_ref[...]
        inv_m = 1.0 / jnp.float32(m_total)
        mean = sum_ref[...] * inv_m
        rows = jax.lax.broadcasted_iota(jnp.int32, (d, d), 0)
        cols = jax.lax.broadcasted_iota(jnp.int32, (d, d), 1)
        eye = jnp.where(rows == cols, jnp.float32(1.0), jnp.float32(0.0))
        outer = jax.lax.dot_general(
            mean, mean, (((1,), (1,)), ((), ())),
            preferred_element_type=jnp.float32,
        )
        sigma = gram * inv_m - outer + EPS * eye
        tr = jnp.sum(jnp.where(rows == cols, sigma, jnp.float32(0.0)))
        r_tr = 1.0 / tr
        sigma_n = sigma * r_tr
        p = eye
        for _ in range(NS_ITERS):
            p2 = jnp.dot(p, p, preferred_element_type=jnp.float32)
            ps = jnp.dot(p, sigma_n, preferred_element_type=jnp.float32)
            p = 1.5 * p - 0.5 * jnp.dot(
                p2, ps, preferred_element_type=jnp.float32
            )
        wm = p * jnp.sqrt(r_tr)
        wm_ref[...] = wm
        wb_ref[...] = jnp.dot(wm, mean, preferred_element_type=jnp.float32)

    @pl.when(j >= nb)
    def _apply():
        for r in range(x_ref.shape[0]):
            o_ref[r] = (
                jnp.dot(
                    wm_ref[...], x_ref[r], preferred_element_type=jnp.float32
                )
                - wb_ref[...]
            )


def kernel(X):
    B, C, L = X.shape
    m_total = B * L
    bb = 4
    nb = B // bb

    x_spec = pl.BlockSpec(
        (bb, C, L),
        lambda j, nb=nb: (jnp.where(j < nb, j, j - nb), 0, 0),
    )
    o_spec = pl.BlockSpec(
        (bb, C, L),
        lambda j, nb=nb: (jnp.where(j < nb, 0, j - nb), 0, 0),
    )

    out = pl.pallas_call(
        functools.partial(_fused_kernel, m_total, nb),
        grid=(2 * nb,),
        in_specs=[x_spec],
        out_specs=o_spec,
        out_shape=jax.ShapeDtypeStruct((B, C, L), jnp.float32),
        scratch_shapes=[
            pltpu.VMEM((C, C), jnp.float32),
            pltpu.VMEM((C, 1), jnp.float32),
            pltpu.VMEM((C, C), jnp.float32),
            pltpu.VMEM((C, 1), jnp.float32),
        ],
        compiler_params=pltpu.CompilerParams(
            dimension_semantics=("arbitrary",),
            vmem_limit_bytes=56 * 1024 * 1024,
        ),
        name="iternorm_fused",
    )(X)

    return out


# speedup vs baseline: 1.1139x; 1.0500x over previous
import functools

import jax
import jax.numpy as jnp
from jax.experimental import pallas as pl
from jax.experimental.pallas import tpu as pltpu

NS_ITERS = 5
EPS = 1e-05


def _fused_kernel(
    m_total, nb, x_ref, o_ref, gram_ref, sum_ref, wm_ref, wb_ref, keep_ref
):
    j = pl.program_id(0)

    @pl.when(j == 0)
    def _init():
        gram_ref[...] = jnp.zeros_like(gram_ref)
        sum_ref[...] = jnp.zeros_like(sum_ref)

    @pl.when(j < nb)
    def _stats():
        gram = gram_ref[...]
        ssum = sum_ref[...]
        for r in range(x_ref.shape[0]):
            x = x_ref[r]
            gram += jax.lax.dot_general(
                x, x, (((1,), (1,)), ((), ())),
                preferred_element_type=jnp.float32,
            )
            ssum += jnp.sum(x, axis=1, keepdims=True)
        gram_ref[...] = gram
        sum_ref[...] = ssum

    @pl.when(j == nb - 3)
    def _keep0():
        keep_ref[0] = x_ref[...]

    @pl.when(j == nb - 2)
    def _keep1():
        keep_ref[1] = x_ref[...]

    @pl.when(j == nb)
    def _compute_wm():
        d = gram_ref.shape[0]
        gram = gram_ref[...]
        inv_m = 1.0 / jnp.float32(m_total)
        mean = sum_ref[...] * inv_m
        rows = jax.lax.broadcasted_iota(jnp.int32, (d, d), 0)
        cols = jax.lax.broadcasted_iota(jnp.int32, (d, d), 1)
        eye = jnp.where(rows == cols, jnp.float32(1.0), jnp.float32(0.0))
        outer = jax.lax.dot_general(
            mean, mean, (((1,), (1,)), ((), ())),
            preferred_element_type=jnp.float32,
        )
        sigma = gram * inv_m - outer + EPS * eye
        tr = jnp.sum(jnp.where(rows == cols, sigma, jnp.float32(0.0)))
        r_tr = 1.0 / tr
        sigma_n = sigma * r_tr
        p = eye
        for _ in range(NS_ITERS):
            p2 = jnp.dot(p, p, preferred_element_type=jnp.float32)
            ps = jnp.dot(p, sigma_n, preferred_element_type=jnp.float32)
            p = 1.5 * p - 0.5 * jnp.dot(
                p2, ps, preferred_element_type=jnp.float32
            )
        wm = p * jnp.sqrt(r_tr)
        wm_ref[...] = wm
        wb_ref[...] = jnp.dot(wm, mean, preferred_element_type=jnp.float32)

    def _emit(load_row):
        wm = wm_ref[...]
        wb = wb_ref[...]
        for r in range(x_ref.shape[0]):
            o_ref[r] = (
                jnp.dot(wm, load_row(r), preferred_element_type=jnp.float32)
                - wb
            )

    @pl.when((j >= nb) & (j != nb + 1) & (j != nb + 2))
    def _apply_streamed():
        _emit(lambda r: x_ref[r])

    @pl.when(j == nb + 1)
    def _apply_keep1():
        _emit(lambda r: keep_ref[1, r])

    @pl.when(j == nb + 2)
    def _apply_keep0():
        _emit(lambda r: keep_ref[0, r])


def kernel(X):
    B, C, L = X.shape
    m_total = B * L
    bb = 4
    nb = B // bb

    x_spec = pl.BlockSpec(
        (bb, C, L),
        lambda j, nb=nb: (
            jnp.where(
                j < nb, j, jnp.where(j <= nb + 2, nb - 1, 2 * nb - 1 - j)
            ),
            0,
            0,
        ),
    )
    o_spec = pl.BlockSpec(
        (bb, C, L),
        lambda j, nb=nb: (jnp.where(j <= nb, nb - 1, 2 * nb - 1 - j), 0, 0),
    )

    out = pl.pallas_call(
        functools.partial(_fused_kernel, m_total, nb),
        grid=(2 * nb,),
        in_specs=[x_spec],
        out_specs=o_spec,
        out_shape=jax.ShapeDtypeStruct((B, C, L), jnp.float32),
        scratch_shapes=[
            pltpu.VMEM((C, C), jnp.float32),
            pltpu.VMEM((C, 1), jnp.float32),
            pltpu.VMEM((C, C), jnp.float32),
            pltpu.VMEM((C, 1), jnp.float32),
            pltpu.VMEM((2, bb, C, L), jnp.float32),
        ],
        compiler_params=pltpu.CompilerParams(
            dimension_semantics=("arbitrary",),
            vmem_limit_bytes=56 * 1024 * 1024,
        ),
        name="iternorm_fused",
    )(X)

    return out


# speedup vs baseline: 1.1170x; 1.0028x over previous
import functools

import jax
import jax.numpy as jnp
from jax.experimental import pallas as pl
from jax.experimental.pallas import tpu as pltpu

NS_ITERS = 5
EPS = 1e-05


def _fused_kernel(
    m_total, nb, x_ref, o_ref, gram_ref, sum_ref, wm_ref, wb_ref, keep_ref
):
    j = pl.program_id(0)

    @pl.when(j == 0)
    def _init():
        gram_ref[...] = jnp.zeros_like(gram_ref)
        sum_ref[...] = jnp.zeros_like(sum_ref)

    @pl.when(j < nb)
    def _stats():
        gram = gram_ref[...]
        ssum = sum_ref[...]
        for r in range(x_ref.shape[0]):
            x = x_ref[r]
            gram += jax.lax.dot_general(
                x, x, (((1,), (1,)), ((), ())),
                preferred_element_type=jnp.float32,
            )
            ssum += jnp.sum(x, axis=1, keepdims=True)
        gram_ref[...] = gram
        sum_ref[...] = ssum

    keep = keep_ref.shape[0]

    @pl.when((j >= nb - 1 - keep) & (j < nb - 1))
    def _keep_block():
        keep_ref[j - (nb - 1 - keep)] = x_ref[...]

    @pl.when(j == nb)
    def _compute_wm():
        d = gram_ref.shape[0]
        gram = gram_ref[...]
        inv_m = 1.0 / jnp.float32(m_total)
        mean = sum_ref[...] * inv_m
        rows = jax.lax.broadcasted_iota(jnp.int32, (d, d), 0)
        cols = jax.lax.broadcasted_iota(jnp.int32, (d, d), 1)
        eye = jnp.where(rows == cols, jnp.float32(1.0), jnp.float32(0.0))
        outer = jax.lax.dot_general(
            mean, mean, (((1,), (1,)), ((), ())),
            preferred_element_type=jnp.float32,
        )
        sigma = gram * inv_m - outer + EPS * eye
        tr = jnp.sum(jnp.where(rows == cols, sigma, jnp.float32(0.0)))
        r_tr = 1.0 / tr
        sigma_n = sigma * r_tr
        p = eye
        for _ in range(NS_ITERS):
            p2 = jnp.dot(p, p, preferred_element_type=jnp.float32)
            ps = jnp.dot(p, sigma_n, preferred_element_type=jnp.float32)
            p = 1.5 * p - 0.5 * jnp.dot(
                p2, ps, preferred_element_type=jnp.float32
            )
        wm = p * jnp.sqrt(r_tr)
        wm_ref[...] = wm
        wb_ref[...] = jnp.dot(wm, mean, preferred_element_type=jnp.float32)

    def _emit(load_row):
        wm = wm_ref[...]
        wb = wb_ref[...]
        for r in range(x_ref.shape[0]):
            o_ref[r] = (
                jnp.dot(wm, load_row(r), preferred_element_type=jnp.float32)
                - wb
            )

    @pl.when((j == nb) | (j > nb + keep))
    def _apply_streamed():
        _emit(lambda r: x_ref[r])

    @pl.when((j > nb) & (j <= nb + keep))
    def _apply_kept():
        _emit(lambda r: keep_ref[keep - (j - nb), r])


def kernel(X):
    B, C, L = X.shape
    m_total = B * L
    bb = 2
    nb = B // bb
    keep = 9

    x_spec = pl.BlockSpec(
        (bb, C, L),
        lambda j, nb=nb, keep=keep: (
            jnp.where(
                j < nb, j, jnp.where(j <= nb + keep, nb - 1, 2 * nb - 1 - j)
            ),
            0,
            0,
        ),
    )
    o_spec = pl.BlockSpec(
        (bb, C, L),
        lambda j, nb=nb: (jnp.where(j <= nb, nb - 1, 2 * nb - 1 - j), 0, 0),
    )

    out = pl.pallas_call(
        functools.partial(_fused_kernel, m_total, nb),
        grid=(2 * nb,),
        in_specs=[x_spec],
        out_specs=o_spec,
        out_shape=jax.ShapeDtypeStruct((B, C, L), jnp.float32),
        scratch_shapes=[
            pltpu.VMEM((C, C), jnp.float32),
            pltpu.VMEM((C, 1), jnp.float32),
            pltpu.VMEM((C, C), jnp.float32),
            pltpu.VMEM((C, 1), jnp.float32),
            pltpu.VMEM((keep, bb, C, L), jnp.float32),
        ],
        compiler_params=pltpu.CompilerParams(
            dimension_semantics=("arbitrary",),
            vmem_limit_bytes=57 * 1024 * 1024,
        ),
        name="iternorm_fused",
    )(X)

    return out
